# baseline (speedup 1.0000x reference)
import jax
import jax.numpy as jnp
from jax import lax
from jax.experimental import pallas as pl
from jax.experimental.pallas import tpu as pltpu
from jax.experimental.pallas import tpu_sc as plsc

BATCH = 4096
HIST = 200
WORD_DIM = 64
AGE_DIM = 32
AGE_VOCAB = 93
OUT_DIM = WORD_DIM + AGE_DIM

NTOK = BATCH * HIST
IDXW = 128
NROWS = NTOK // IDXW
NWORKERS = 32
ROWS_PER_W = NROWS // NWORKERS
RPC = 4
CHUNK = RPC * IDXW
NCHUNKS = ROWS_PER_W // RPC


def _body(widx_hbm, aidx_hbm, wtab_hbm, atab_hbm, out_hbm,
          widx_v, aidx_v, wrows_v, a0_v, a1_v, atab_v,
          isem, gsem0, gsem1, wsem):
    cid = lax.axis_index("c")
    sid = lax.axis_index("s")
    wid = sid * 2 + cid
    row_base = wid * ROWS_PER_W
    gsems = (gsem0, gsem1)
    arows = (a0_v, a1_v)

    pltpu.sync_copy(atab_hbm, atab_v)

    def start_idx(i, s):
        r = row_base + i * RPC
        pltpu.async_copy(widx_hbm.at[pl.ds(r, RPC)], widx_v.at[s], isem)
        pltpu.async_copy(aidx_hbm.at[pl.ds(r * IDXW, CHUNK)], aidx_v.at[s], isem)

    def wait_idx(s):
        pltpu.make_async_copy(
            widx_hbm.at[pl.ds(0, RPC)], widx_v.at[s], isem).wait()
        pltpu.make_async_copy(
            aidx_hbm.at[pl.ds(0, CHUNK)], aidx_v.at[s], isem).wait()

    def fire_word(s):
        for j in range(RPC):
            pltpu.async_copy(wtab_hbm.at[widx_v.at[s, j]],
                             wrows_v.at[s, pl.ds(j * IDXW, IDXW)], gsems[s])

    def wait_word(s):
        pltpu.make_async_copy(
            wtab_hbm.at[pl.ds(0, CHUNK)], wrows_v.at[s], gsems[s]).wait()

    iota16 = lax.iota(jnp.int32, 16)
    cvecs = [jnp.full((16,), c, jnp.int32) for c in range(AGE_DIM)]

    def age_compute(s):
        arow_ref = arows[s]

        def grp(g, carry):
            av = aidx_v[s, pl.ds(g * 16, 16)]
            base = av * AGE_DIM
            rows = g * 16 + iota16
            for c in range(AGE_DIM):
                vals = plsc.load_gather(atab_v, [base + cvecs[c]])
                plsc.store_scatter(arow_ref, [rows, cvecs[c]], vals)
            return carry

        lax.fori_loop(0, CHUNK // 16, grp, 0)

    def start_write(i, s):
        base = (row_base + i * RPC) * IDXW
        pltpu.async_copy(
            wrows_v.at[s],
            out_hbm.at[pl.ds(base, CHUNK), pl.ds(0, WORD_DIM)], wsem)
        pltpu.async_copy(
            arows[s],
            out_hbm.at[pl.ds(base, CHUNK), pl.ds(WORD_DIM, AGE_DIM)], wsem)

    def wait_write():
        pltpu.make_async_copy(
            wrows_v.at[0],
            out_hbm.at[pl.ds(0, CHUNK), pl.ds(0, WORD_DIM)], wsem).wait()
        pltpu.make_async_copy(
            a0_v,
            out_hbm.at[pl.ds(0, CHUNK), pl.ds(WORD_DIM, AGE_DIM)], wsem).wait()

    start_idx(0, 0)
    wait_idx(0)
    fire_word(0)
    age_compute(0)
    start_idx(1, 1)
    wait_idx(1)
    fire_word(1)
    age_compute(1)
    wait_word(0)
    start_write(0, 0)
    start_idx(2, 0)
    wait_idx(0)

    def step2(t, carry):
        for k in range(2):
            i = 2 + 2 * t + k
            s, p = k, 1 - k
            wait_write()
            fire_word(s)
            age_compute(s)
            wait_word(p)
            start_write(i - 1, p)
            start_idx(jnp.minimum(i + 1, NCHUNKS - 1), p)
            wait_idx(p)
        return carry

    lax.fori_loop(0, (NCHUNKS - 2) // 2, step2, 0)

    last = (NCHUNKS - 1) % 2
    wait_word(last)
    start_write(NCHUNKS - 1, last)
    wait_write()
    wait_write()


@jax.jit
def _embed(widx, aidx, word_table, age_table_flat):
    kern = pl.kernel(
        _body,
        out_type=jax.ShapeDtypeStruct((NTOK, OUT_DIM), jnp.float32),
        mesh=plsc.VectorSubcoreMesh(core_axis_name="c", subcore_axis_name="s"),
        scratch_types=[
            pltpu.VMEM((2, RPC, IDXW), jnp.int32),
            pltpu.VMEM((2, CHUNK), jnp.int32),
            pltpu.VMEM((2, CHUNK, WORD_DIM), jnp.float32),
            pltpu.VMEM((CHUNK, AGE_DIM), jnp.float32),
            pltpu.VMEM((CHUNK, AGE_DIM), jnp.float32),
            pltpu.VMEM((AGE_VOCAB * AGE_DIM,), jnp.float32),
            pltpu.SemaphoreType.DMA,
            pltpu.SemaphoreType.DMA,
            pltpu.SemaphoreType.DMA,
            pltpu.SemaphoreType.DMA,
        ],
        compiler_params=pltpu.CompilerParams(use_tc_tiling_on_sc=False,
                                             needs_layout_passes=False),
    )
    return kern(widx, aidx, word_table, age_table_flat)


def kernel(word, age, word_table, age_table):
    widx = word.astype(jnp.int32).reshape(NROWS, IDXW)
    aidx = age.astype(jnp.int32).reshape(NTOK)
    out = _embed(widx, aidx, word_table, age_table.reshape(-1))
    return out.reshape(BATCH, HIST, OUT_DIM)


# speedup vs baseline: 1.0894x; 1.0894x over previous
import jax
import jax.numpy as jnp
from jax import lax
from jax.experimental import pallas as pl
from jax.experimental.pallas import tpu as pltpu
from jax.experimental.pallas import tpu_sc as plsc

BATCH = 4096
HIST = 200
WORD_DIM = 64
AGE_DIM = 32
OUT_DIM = WORD_DIM + AGE_DIM

NTOK = BATCH * HIST
IDXW = 128
NROWS = NTOK // IDXW
NWORKERS = 32
ROWS_PER_W = NROWS // NWORKERS
RPC = 2
CHUNK = RPC * IDXW
NCHUNKS = ROWS_PER_W // RPC
NSLOT = 4


def _body(widx_hbm, aidx_hbm, wtab_hbm, atab_hbm, out_hbm,
          widx_v, aidx_v, wrows_v, arows_v,
          isem0, isem1, isem2, isem3,
          gsem0, gsem1, gsem2, gsem3,
          wsem0, wsem1, wsem2, wsem3):
    cid = lax.axis_index("c")
    sid = lax.axis_index("s")
    wid = sid * 2 + cid
    row_base = wid * ROWS_PER_W
    isems = (isem0, isem1, isem2, isem3)
    gsems = (gsem0, gsem1, gsem2, gsem3)
    wsems = (wsem0, wsem1, wsem2, wsem3)

    def start_idx(i, s):
        r = row_base + i * RPC
        pltpu.async_copy(widx_hbm.at[pl.ds(r, RPC)], widx_v.at[s], isems[s])
        pltpu.async_copy(aidx_hbm.at[pl.ds(r, RPC)], aidx_v.at[s], isems[s])

    def wait_idx(s):
        pltpu.make_async_copy(
            widx_hbm.at[pl.ds(0, RPC)], widx_v.at[s], isems[s]).wait()
        pltpu.make_async_copy(
            aidx_hbm.at[pl.ds(0, RPC)], aidx_v.at[s], isems[s]).wait()

    def fire_gathers(s):
        for j in range(RPC):
            pltpu.async_copy(wtab_hbm.at[widx_v.at[s, j]],
                             wrows_v.at[s, j], gsems[s])
            pltpu.async_copy(atab_hbm.at[aidx_v.at[s, j]],
                             arows_v.at[s, j], gsems[s])

    def wait_gathers(s):
        pltpu.make_async_copy(
            out_hbm.at[pl.ds(0, RPC), :, pl.ds(0, WORD_DIM)],
            wrows_v.at[s], gsems[s]).wait()
        pltpu.make_async_copy(
            out_hbm.at[pl.ds(0, RPC), :, pl.ds(WORD_DIM, AGE_DIM)],
            arows_v.at[s], gsems[s]).wait()

    def start_write(i, s):
        r = row_base + i * RPC
        pltpu.async_copy(
            wrows_v.at[s],
            out_hbm.at[pl.ds(r, RPC), :, pl.ds(0, WORD_DIM)], wsems[s])
        pltpu.async_copy(
            arows_v.at[s],
            out_hbm.at[pl.ds(r, RPC), :, pl.ds(WORD_DIM, AGE_DIM)], wsems[s])

    def wait_write(s):
        pltpu.make_async_copy(
            wrows_v.at[s],
            out_hbm.at[pl.ds(0, RPC), :, pl.ds(0, WORD_DIM)], wsems[s]).wait()
        pltpu.make_async_copy(
            arows_v.at[s],
            out_hbm.at[pl.ds(0, RPC), :, pl.ds(WORD_DIM, AGE_DIM)],
            wsems[s]).wait()

    start_idx(0, 0)
    start_idx(1, 1)
    wait_idx(0)
    fire_gathers(0)
    start_idx(2, 2)
    wait_idx(1)
    fire_gathers(1)
    start_idx(3, 3)
    wait_idx(2)
    fire_gathers(2)
    wait_gathers(0)
    start_write(0, 0)
    start_idx(4, 0)
    wait_idx(3)
    fire_gathers(3)
    wait_gathers(1)
    start_write(1, 1)
    start_idx(5, 1)

    def step4(t, carry):
        for k in range(4):
            i = 4 + 4 * t + k
            s = k
            wait_write(s)
            wait_idx(s)
            fire_gathers(s)
            sp = (k + 2) % 4
            wait_gathers(sp)
            start_write(i - 2, sp)
            start_idx(jnp.minimum(i + 2, NCHUNKS - 1), sp)
        return carry

    lax.fori_loop(0, (NCHUNKS - 4) // 4, step4, 0)

    wait_gathers(2)
    start_write(NCHUNKS - 2, 2)
    wait_gathers(3)
    start_write(NCHUNKS - 1, 3)
    wait_write(0)
    wait_write(1)
    wait_write(2)
    wait_write(3)
    wait_idx(0)
    wait_idx(1)


@jax.jit
def _embed(widx, aidx, word_table, age_table):
    kern = pl.kernel(
        _body,
        out_type=jax.ShapeDtypeStruct((NROWS, IDXW, OUT_DIM), jnp.float32),
        mesh=plsc.VectorSubcoreMesh(core_axis_name="c", subcore_axis_name="s"),
        scratch_types=[
            pltpu.VMEM((NSLOT, RPC, IDXW), jnp.int32),
            pltpu.VMEM((NSLOT, RPC, IDXW), jnp.int32),
            pltpu.VMEM((NSLOT, RPC, IDXW, WORD_DIM), jnp.float32),
            pltpu.VMEM((NSLOT, RPC, IDXW, AGE_DIM), jnp.float32),
        ] + [pltpu.SemaphoreType.DMA] * 12,
        compiler_params=pltpu.CompilerParams(use_tc_tiling_on_sc=False),
    )
    return kern(widx, aidx, word_table, age_table)


def kernel(word, age, word_table, age_table):
    widx = word.astype(jnp.int32).reshape(NROWS, IDXW)
    aidx = age.astype(jnp.int32).reshape(NROWS, IDXW)
    out = _embed(widx, aidx, word_table, age_table)
    return out.reshape(BATCH, HIST, OUT_DIM)
